# baseline (speedup 1.0000x reference)
import jax
import jax.numpy as jnp
from jax.experimental import pallas as pl
from jax.experimental.pallas import tpu as pltpu

_CHUNKS = 16


def _copy_kernel(src_hbm, dst_hbm, buf, *sems):
    rows = src_hbm.shape[0] // _CHUNKS
    in_sems, out_sems = sems[:_CHUNKS], sems[_CHUNKS:]
    ins = [
        pltpu.make_async_copy(
            src_hbm.at[pl.ds(i * rows, rows), :],
            buf.at[pl.ds(i * rows, rows), :],
            in_sems[i],
        )
        for i in range(_CHUNKS)
    ]
    outs = [
        pltpu.make_async_copy(
            buf.at[pl.ds(i * rows, rows), :],
            dst_hbm.at[pl.ds(i * rows, rows), :],
            out_sems[i],
        )
        for i in range(_CHUNKS)
    ]
    for c in ins:
        c.start()
    for i in range(_CHUNKS):
        ins[i].wait()
        outs[i].start()
    for c in outs:
        c.wait()


def kernel(features, blocks, cluster_centers, W1, b1, W2, b2, epoch, max_epochs):
    N, L = blocks.shape
    return pl.pallas_call(
        _copy_kernel,
        in_specs=[pl.BlockSpec(memory_space=pl.ANY)],
        out_specs=pl.BlockSpec(memory_space=pl.ANY),
        out_shape=jax.ShapeDtypeStruct((N, L), blocks.dtype),
        scratch_shapes=[pltpu.MemorySpace.VMEM((N, L), blocks.dtype)]
        + [pltpu.SemaphoreType.DMA] * (2 * _CHUNKS),
    )(blocks)


# speedup vs baseline: 1.0556x; 1.0556x over previous
import jax
import jax.numpy as jnp
from jax.experimental import pallas as pl
from jax.experimental.pallas import tpu as pltpu

_CHUNKS = 4


def _copy_kernel(src_hbm, dst_hbm, buf, *sems):
    rows = src_hbm.shape[0] // _CHUNKS
    in_sems, out_sems = sems[:_CHUNKS], sems[_CHUNKS:]
    ins = [
        pltpu.make_async_copy(
            src_hbm.at[pl.ds(i * rows, rows), :],
            buf.at[pl.ds(i * rows, rows), :],
            in_sems[i],
        )
        for i in range(_CHUNKS)
    ]
    outs = [
        pltpu.make_async_copy(
            buf.at[pl.ds(i * rows, rows), :],
            dst_hbm.at[pl.ds(i * rows, rows), :],
            out_sems[i],
        )
        for i in range(_CHUNKS)
    ]
    for c in ins:
        c.start()
    for i in range(_CHUNKS):
        ins[i].wait()
        outs[i].start()
    for c in outs:
        c.wait()


def kernel(features, blocks, cluster_centers, W1, b1, W2, b2, epoch, max_epochs):
    N, L = blocks.shape
    return pl.pallas_call(
        _copy_kernel,
        in_specs=[pl.BlockSpec(memory_space=pl.ANY)],
        out_specs=pl.BlockSpec(memory_space=pl.ANY),
        out_shape=jax.ShapeDtypeStruct((N, L), blocks.dtype),
        scratch_shapes=[pltpu.MemorySpace.VMEM((N, L), blocks.dtype)]
        + [pltpu.SemaphoreType.DMA] * (2 * _CHUNKS),
    )(blocks)
